# baseline (speedup 1.0000x reference)
import functools
import math

import jax
import jax.numpy as jnp
from jax import lax
from jax.experimental import pallas as pl
from jax.experimental.pallas import tpu as pltpu
from jax.experimental.pallas import tpu_sc as plsc

N, E, D, H = 10000, 320000, 128, 8
DK = D // H

NC, NS, L = 2, 16, 16
NW = NC * NS
EPW = E // NW
B = 40
CHUNKS = EPW // B
TOT_BLKS = E // B
W = D + 16
RPT = 624
TAIL = N - NS * RPT


def _tc_qkv(xb, wb, bb, oq, okv):
    r = jnp.dot(xb[...], wb[...], preferred_element_type=jnp.float32) + bb[...]
    oq[...] = r[:, :D]
    okv[...] = r[:, D:3 * D]


def _sc_edge(q_hbm, kv_hbm, meta_hbm, z_hbm, out_hbm,
             acc, mb0, qg0, kvg0, mb1, qg1, kvg1, ob, sem0, sem1):
    cid = lax.axis_index("c")
    sid = lax.axis_index("s")
    wid = cid * NS + sid

    @pl.when(sid == 0)
    def _():
        pltpu.sync_copy(z_hbm, acc)

    plsc.subcore_barrier()

    lane = lax.broadcasted_iota(jnp.int32, (L,), 0)
    bufs = ((mb0, qg0, kvg0, sem0), (mb1, qg1, kvg1, sem1))
    base = wid * CHUNKS
    last = TOT_BLKS - 1

    def fetch(blk, buf):
        mb, qg, kvg, sem = buf
        pltpu.sync_copy(meta_hbm.at[blk], mb)
        pltpu.async_copy(q_hbm.at[mb.at[1]], qg, sem)
        pltpu.async_copy(kv_hbm.at[mb.at[0]], kvg, sem)

    def drain(buf):
        mb, qg, kvg, sem = buf
        pltpu.make_async_copy(q_hbm.at[mb.at[1]], qg, sem).wait()
        pltpu.make_async_copy(kv_hbm.at[mb.at[0]], kvg, sem).wait()

    def cfull(c):
        return jnp.full((L,), c, jnp.int32)

    evecs = []
    for g0 in range(0, B, L):
        if g0 + L <= B:
            evecs.append(g0 + lane)
        else:
            evecs.append(g0 + (lane & (B - g0 - 1)))

    def compute(buf):
        mb, qg, kvg, _ = buf
        for ev in evecs:
            ewi = plsc.load_gather(mb, [cfull(2), ev])
            ew_v = plsc.bitcast(ewi, jnp.float32)

            def head_body(h, carry):
                c0 = jnp.full((L,), h * DK, jnp.int32)
                accs = [jnp.zeros((L,), jnp.float32) for _ in range(4)]
                for dd in range(DK):
                    qc = plsc.load_gather(qg, [ev, c0 + dd])
                    kc = plsc.load_gather(kvg, [ev, c0 + dd])
                    accs[dd % 4] = accs[dd % 4] + qc * kc
                s = (accs[0] + accs[1]) + (accs[2] + accs[3])
                ex_h = jnp.exp(s * ew_v)
                plsc.store_scatter(ob, [ev, cfull(D) + h], ex_h)
                for dd in range(DK):
                    vc = plsc.load_gather(kvg, [ev, c0 + (D + dd)])
                    plsc.store_scatter(ob, [ev, c0 + dd], vc * ex_h)
                return carry

            lax.fori_loop(0, H, head_body, 0)

        pltpu.sync_copy(ob, acc.at[mb.at[1]], add=True)

    zl = jnp.zeros((L,), jnp.float32)
    for ev in evecs:
        for c in range(D + H, W):
            plsc.store_scatter(ob, [ev, cfull(c)], zl)

    fetch(base, bufs[0])

    def chunk2_body(ci2, carry):
        ci = ci2 * 2
        for par in range(2):
            cur, nxt = bufs[par], bufs[1 - par]
            fetch(jnp.minimum(base + ci + par + 1, last), nxt)
            drain(cur)
            compute(cur)
        return carry

    lax.fori_loop(0, CHUNKS // 2, chunk2_body, 0)
    drain(bufs[0])
    plsc.subcore_barrier()
    pltpu.sync_copy(acc.at[pl.ds(sid * RPT, RPT)],
                    out_hbm.at[cid, pl.ds(sid * RPT, RPT)])

    @pl.when(sid == 0)
    def _():
        pltpu.sync_copy(acc.at[pl.ds(NS * RPT, TAIL)],
                        out_hbm.at[cid, pl.ds(NS * RPT, TAIL)])


def _tc_out(accb, xb, wab, bab, alb, ob):
    a0 = accb[0]
    a1 = accb[1]
    num = a0[:, :D] + a1[:, :D]
    den = a0[:, D:D + L] + a1[:, D:D + L]
    recip = 1.0 / jnp.maximum(den, 1e-12)
    col = lax.broadcasted_iota(jnp.int32, (L, D), 1) // DK
    row = lax.broadcasted_iota(jnp.int32, (L, D), 0)
    sel = (col == row).astype(jnp.float32)
    rep = jnp.dot(recip, sel, preferred_element_type=jnp.float32)
    t = num * rep
    trans = jnp.dot(t, wab[...], preferred_element_type=jnp.float32) + bab[...]
    al = alb[...]
    ob[...] = trans * al + xb[...] * (1.0 - al)


def kernel(x, edge_weight, Wk, bk, Wq, bq, Wv, bv, Wa, ba, rel_att, rel_msg,
           rel_pri, skip, edge_index):
    f32 = jnp.float32
    Batt = jax.scipy.linalg.block_diag(*[rel_att[h] for h in range(H)])
    Bmsg = jax.scipy.linalg.block_diag(*[rel_msg[h] for h in range(H)])
    s_vec = jnp.repeat(rel_pri, DK) / math.sqrt(DK)
    Wq_eff = Wq.T * s_vec[None, :]
    bq_eff = bq * s_vec
    Wk_eff = Wk.T @ Batt
    bk_eff = bk @ Batt
    Wv_eff = Wv.T @ Bmsg
    bv_eff = bv @ Bmsg
    Wcat = jnp.concatenate([Wq_eff, Wk_eff, Wv_eff], axis=1)
    bcat = jnp.concatenate([bq_eff, bk_eff, bv_eff])[None, :]
    alpha = jnp.full((1, D), jax.nn.sigmoid(skip), f32)

    BN = 1000
    grid = N // BN
    q, kv = pl.pallas_call(
        _tc_qkv,
        grid=(grid,),
        in_specs=[
            pl.BlockSpec((BN, D), lambda i: (i, 0)),
            pl.BlockSpec((D, 3 * D), lambda i: (0, 0)),
            pl.BlockSpec((1, 3 * D), lambda i: (0, 0)),
        ],
        out_specs=[pl.BlockSpec((BN, D), lambda i: (i, 0)),
                   pl.BlockSpec((BN, 2 * D), lambda i: (i, 0))],
        out_shape=[jax.ShapeDtypeStruct((N, D), f32),
                   jax.ShapeDtypeStruct((N, 2 * D), f32)],
    )(x, Wcat, bcat)

    src_b = edge_index[0].reshape(-1, B)
    dst_b = edge_index[1].reshape(-1, B)
    ew_b = lax.bitcast_convert_type(edge_weight, jnp.int32).reshape(-1, B)
    meta = jnp.stack([src_b, dst_b, ew_b, jnp.zeros_like(src_b)], axis=1)

    zeros = jnp.zeros((N, W), f32)
    sc = pl.kernel(
        _sc_edge,
        out_type=jax.ShapeDtypeStruct((NC, N, W), f32),
        mesh=plsc.VectorSubcoreMesh(core_axis_name="c", subcore_axis_name="s"),
        compiler_params=pltpu.CompilerParams(
            needs_layout_passes=False, use_tc_tiling_on_sc=False),
        scratch_types=[
            pltpu.VMEM_SHARED((N, W), f32),
            pltpu.VMEM((4, B), jnp.int32),
            pltpu.VMEM((B, D), f32),
            pltpu.VMEM((B, 2 * D), f32),
            pltpu.VMEM((4, B), jnp.int32),
            pltpu.VMEM((B, D), f32),
            pltpu.VMEM((B, 2 * D), f32),
            pltpu.VMEM((B, W), f32),
            pltpu.SemaphoreType.DMA,
            pltpu.SemaphoreType.DMA,
        ],
    )
    acc_out = sc(q, kv, meta, zeros)

    out = pl.pallas_call(
        _tc_out,
        grid=(grid,),
        in_specs=[
            pl.BlockSpec((NC, BN, W), lambda i: (0, i, 0)),
            pl.BlockSpec((BN, D), lambda i: (i, 0)),
            pl.BlockSpec((D, D), lambda i: (0, 0)),
            pl.BlockSpec((1, D), lambda i: (0, 0)),
            pl.BlockSpec((1, D), lambda i: (0, 0)),
        ],
        out_specs=pl.BlockSpec((BN, D), lambda i: (i, 0)),
        out_shape=jax.ShapeDtypeStruct((N, D), f32),
    )(acc_out, x, Wa.T, ba[None, :], alpha)
    return out


# speedup vs baseline: 5.6940x; 5.6940x over previous
import functools
import math

import jax
import jax.numpy as jnp
from jax import lax
from jax.experimental import pallas as pl
from jax.experimental.pallas import tpu as pltpu
from jax.experimental.pallas import tpu_sc as plsc

N, E, D, H = 10000, 320000, 128, 8
DK = D // H

NC, NS, L = 2, 16, 16
NW = NC * NS
EPW = E // NW
B = 40
CHUNKS = EPW // B
TOT_BLKS = E // B
W = D + 16
RPT = 624
TAIL = N - NS * RPT


def _tc_qkv(xb, wb, bb, oq, okv):
    r = jnp.dot(xb[...], wb[...], preferred_element_type=jnp.float32) + bb[...]
    oq[...] = r[:, :D]
    okv[...] = r[:, D:3 * D]


def _sc_edge(q_hbm, kv_hbm, meta_hbm, z_hbm, out_hbm,
             acc, mb0, qg0, kvg0, mb1, qg1, kvg1, ob, sem0, sem1):
    cid = lax.axis_index("c")
    sid = lax.axis_index("s")
    wid = cid * NS + sid

    @pl.when(sid == 0)
    def _():
        pltpu.sync_copy(z_hbm, acc)

    plsc.subcore_barrier()

    lane = lax.broadcasted_iota(jnp.int32, (L,), 0)
    bufs = ((mb0, qg0, kvg0, sem0), (mb1, qg1, kvg1, sem1))
    base = wid * CHUNKS
    last = TOT_BLKS - 1

    def fetch(blk, buf):
        mb, qg, kvg, sem = buf
        pltpu.sync_copy(meta_hbm.at[blk], mb)
        pltpu.async_copy(q_hbm.at[mb.at[1]], qg, sem)
        pltpu.async_copy(kv_hbm.at[mb.at[0]], kvg, sem)

    def drain(buf):
        mb, qg, kvg, sem = buf
        pltpu.make_async_copy(q_hbm.at[mb.at[1]], qg, sem).wait()
        pltpu.make_async_copy(kv_hbm.at[mb.at[0]], kvg, sem).wait()

    def compute(buf):
        mb, qg, kvg, _ = buf

        @plsc.parallel_loop(0, B, 1, unroll=4)
        def edge_body(e):
            ewi = plsc.load_gather(
                mb, [jnp.full((L,), 2, jnp.int32), jnp.full((L,), e, jnp.int32)])
            ew_e = plsc.bitcast(ewi, jnp.float32)
            parts = []
            for h in range(H):
                qv = qg[e, pl.ds(h * DK, DK)]
                kv = kvg[e, pl.ds(h * DK, DK)]
                s = jnp.sum(qv * kv)
                parts.append(jnp.where(lane == h, s, 0.0))
            while len(parts) > 1:
                parts = [a + b for a, b in zip(parts[::2], parts[1::2])]
            ex = jnp.exp(parts[0] * ew_e)
            ob[e, pl.ds(D, L)] = ex
            e_full = jnp.full((L,), e, jnp.int32)
            for h in range(H):
                exh = plsc.load_gather(
                    ob, [e_full, jnp.full((L,), D + h, jnp.int32)])
                ob[e, pl.ds(h * DK, DK)] = kvg[e, pl.ds(D + h * DK, DK)] * exh

        pltpu.sync_copy(ob, acc.at[mb.at[1]], add=True)

    fetch(base, bufs[0])

    def chunk2_body(ci2, carry):
        ci = ci2 * 2
        for par in range(2):
            cur, nxt = bufs[par], bufs[1 - par]
            fetch(jnp.minimum(base + ci + par + 1, last), nxt)
            drain(cur)
            compute(cur)
        return carry

    lax.fori_loop(0, CHUNKS // 2, chunk2_body, 0)
    drain(bufs[0])
    plsc.subcore_barrier()
    pltpu.sync_copy(acc.at[pl.ds(sid * RPT, RPT)],
                    out_hbm.at[cid, pl.ds(sid * RPT, RPT)])

    @pl.when(sid == 0)
    def _():
        pltpu.sync_copy(acc.at[pl.ds(NS * RPT, TAIL)],
                        out_hbm.at[cid, pl.ds(NS * RPT, TAIL)])


def _tc_out(accb, xb, wab, bab, alb, ob):
    a0 = accb[0]
    a1 = accb[1]
    num = a0[:, :D] + a1[:, :D]
    den = a0[:, D:D + L] + a1[:, D:D + L]
    recip = 1.0 / jnp.maximum(den, 1e-12)
    col = lax.broadcasted_iota(jnp.int32, (L, D), 1) // DK
    row = lax.broadcasted_iota(jnp.int32, (L, D), 0)
    sel = (col == row).astype(jnp.float32)
    rep = jnp.dot(recip, sel, preferred_element_type=jnp.float32)
    t = num * rep
    trans = jnp.dot(t, wab[...], preferred_element_type=jnp.float32) + bab[...]
    al = alb[...]
    ob[...] = trans * al + xb[...] * (1.0 - al)


def kernel(x, edge_weight, Wk, bk, Wq, bq, Wv, bv, Wa, ba, rel_att, rel_msg,
           rel_pri, skip, edge_index):
    f32 = jnp.float32
    Batt = jax.scipy.linalg.block_diag(*[rel_att[h] for h in range(H)])
    Bmsg = jax.scipy.linalg.block_diag(*[rel_msg[h] for h in range(H)])
    s_vec = jnp.repeat(rel_pri, DK) / math.sqrt(DK)
    Wq_eff = Wq.T * s_vec[None, :]
    bq_eff = bq * s_vec
    Wk_eff = Wk.T @ Batt
    bk_eff = bk @ Batt
    Wv_eff = Wv.T @ Bmsg
    bv_eff = bv @ Bmsg
    Wcat = jnp.concatenate([Wq_eff, Wk_eff, Wv_eff], axis=1)
    bcat = jnp.concatenate([bq_eff, bk_eff, bv_eff])[None, :]
    alpha = jnp.full((1, D), jax.nn.sigmoid(skip), f32)

    BN = 1000
    grid = N // BN
    q, kv = pl.pallas_call(
        _tc_qkv,
        grid=(grid,),
        in_specs=[
            pl.BlockSpec((BN, D), lambda i: (i, 0)),
            pl.BlockSpec((D, 3 * D), lambda i: (0, 0)),
            pl.BlockSpec((1, 3 * D), lambda i: (0, 0)),
        ],
        out_specs=[pl.BlockSpec((BN, D), lambda i: (i, 0)),
                   pl.BlockSpec((BN, 2 * D), lambda i: (i, 0))],
        out_shape=[jax.ShapeDtypeStruct((N, D), f32),
                   jax.ShapeDtypeStruct((N, 2 * D), f32)],
    )(x, Wcat, bcat)

    src_b = edge_index[0].reshape(-1, B)
    dst_b = edge_index[1].reshape(-1, B)
    ew_b = lax.bitcast_convert_type(edge_weight, jnp.int32).reshape(-1, B)
    meta = jnp.stack([src_b, dst_b, ew_b, jnp.zeros_like(src_b)], axis=1)

    zeros = jnp.zeros((N, W), f32)
    sc = pl.kernel(
        _sc_edge,
        out_type=jax.ShapeDtypeStruct((NC, N, W), f32),
        mesh=plsc.VectorSubcoreMesh(core_axis_name="c", subcore_axis_name="s"),
        compiler_params=pltpu.CompilerParams(
            needs_layout_passes=False, use_tc_tiling_on_sc=False),
        scratch_types=[
            pltpu.VMEM_SHARED((N, W), f32),
            pltpu.VMEM((4, B), jnp.int32),
            pltpu.VMEM((B, D), f32),
            pltpu.VMEM((B, 2 * D), f32),
            pltpu.VMEM((4, B), jnp.int32),
            pltpu.VMEM((B, D), f32),
            pltpu.VMEM((B, 2 * D), f32),
            pltpu.VMEM((B, W), f32),
            pltpu.SemaphoreType.DMA,
            pltpu.SemaphoreType.DMA,
        ],
    )
    acc_out = sc(q, kv, meta, zeros)

    out = pl.pallas_call(
        _tc_out,
        grid=(grid,),
        in_specs=[
            pl.BlockSpec((NC, BN, W), lambda i: (0, i, 0)),
            pl.BlockSpec((BN, D), lambda i: (i, 0)),
            pl.BlockSpec((D, D), lambda i: (0, 0)),
            pl.BlockSpec((1, D), lambda i: (0, 0)),
            pl.BlockSpec((1, D), lambda i: (0, 0)),
        ],
        out_specs=pl.BlockSpec((BN, D), lambda i: (i, 0)),
        out_shape=jax.ShapeDtypeStruct((N, D), f32),
    )(acc_out, x, Wa.T, ba[None, :], alpha)
    return out
